# baseline (speedup 1.0000x reference)
import functools

import jax
import jax.numpy as jnp
from jax import lax
from jax.experimental import pallas as pl
from jax.experimental.pallas import tpu as pltpu
from jax.experimental.pallas import tpu_sc as plsc

N_NODES = 10000
N_EDGES = 320000
D_IN = 128

NPAD = 10240
NW = 32
CH = 128
GB = 4
EPAD = 327680
NG = EPAD // (NW * GB * CH)
RPT = NPAD // 16

_f32 = jnp.float32


def _tc1_body(x_ref, wl_ref, wr_ref, b_ref, p_ref, s_ref):
    x = x_ref[...]
    p_ref[...] = jnp.dot(x, wl_ref[...], preferred_element_type=_f32)
    s_ref[...] = jnp.dot(x, wr_ref[...], preferred_element_type=_f32) + b_ref[...]


def _tc1(x_pad, W_l, W_r, b):
    h = W_l.shape[1]
    return pl.pallas_call(
        _tc1_body,
        out_shape=[
            jax.ShapeDtypeStruct((NPAD, h), _f32),
            jax.ShapeDtypeStruct((NPAD, h), _f32),
        ],
    )(x_pad, W_l, W_r, b.reshape(1, h))


def _tc2_body(pa_ref, pb_ref, da_ref, db_ref, s1_ref, wl_ref, wr_ref, b_ref,
              p2_ref, s2_ref, inv_ref):
    inv = 1.0 / jnp.maximum(da_ref[...] + db_ref[...], 1.0)
    h = jnp.maximum((pa_ref[...] + pb_ref[...]) * inv + s1_ref[...], 0.0)
    p2_ref[...] = jnp.dot(h, wl_ref[...], preferred_element_type=_f32)
    s2_ref[...] = jnp.dot(h, wr_ref[...], preferred_element_type=_f32) + b_ref[...]
    inv_ref[...] = inv


def _tc2(part, deg2, s1, W_l, W_r, b):
    h1 = s1.shape[1]
    h2 = W_l.shape[1]
    return pl.pallas_call(
        _tc2_body,
        grid=(1,),
        in_specs=[
            pl.BlockSpec((NPAD, h1), lambda i: (0, 0)),
            pl.BlockSpec((NPAD, h1), lambda i: (1, 0)),
            pl.BlockSpec((NPAD, 1), lambda i: (0, 0)),
            pl.BlockSpec((NPAD, 1), lambda i: (1, 0)),
            pl.BlockSpec((NPAD, h1), lambda i: (0, 0)),
            pl.BlockSpec((h1, h2), lambda i: (0, 0)),
            pl.BlockSpec((h1, h2), lambda i: (0, 0)),
            pl.BlockSpec((1, h2), lambda i: (0, 0)),
        ],
        out_specs=[
            pl.BlockSpec((NPAD, h2), lambda i: (0, 0)),
            pl.BlockSpec((NPAD, h2), lambda i: (0, 0)),
            pl.BlockSpec((NPAD, 1), lambda i: (0, 0)),
        ],
        out_shape=[
            jax.ShapeDtypeStruct((NPAD, h2), _f32),
            jax.ShapeDtypeStruct((NPAD, h2), _f32),
            jax.ShapeDtypeStruct((NPAD, 1), _f32),
        ],
    )(part, part, deg2, deg2, s1, W_l, W_r, b.reshape(1, h2))


def _tc3_body(pa_ref, pb_ref, inv_ref, s2_ref, w_ref, out_ref):
    h = jnp.maximum((pa_ref[...] + pb_ref[...]) * inv_ref[...] + s2_ref[...], 0.0)
    out_ref[...] = jnp.dot(h, w_ref[...], preferred_element_type=_f32)


def _tc3(part, inv, s2, w):
    h2 = s2.shape[1]
    dout = w.shape[1]
    return pl.pallas_call(
        _tc3_body,
        grid=(1,),
        in_specs=[
            pl.BlockSpec((NPAD, h2), lambda i: (0, 0)),
            pl.BlockSpec((NPAD, h2), lambda i: (1, 0)),
            pl.BlockSpec((NPAD, 1), lambda i: (0, 0)),
            pl.BlockSpec((NPAD, h2), lambda i: (0, 0)),
            pl.BlockSpec((h2, dout), lambda i: (0, 0)),
        ],
        out_specs=pl.BlockSpec((NPAD, dout), lambda i: (0, 0)),
        out_shape=jax.ShapeDtypeStruct((NPAD, dout), _f32),
    )(part, part, inv, s2, w)


def _sc_mesh():
    return plsc.VectorSubcoreMesh(core_axis_name="c", subcore_axis_name="s")


def _seg_loop(tab_sh, acc_sh, src_v, dst_v, bufs, sems,
              deg_sh=None, ones_v=None, dsem=None):
    pltpu.async_copy(tab_sh.at[src_v.at[0]], bufs[0], sems[0])
    pltpu.async_copy(tab_sh.at[src_v.at[1]], bufs[1], sems[1])
    for g in range(NG):
        b = g % 3
        pltpu.make_async_copy(tab_sh.at[src_v.at[g]], bufs[b], sems[b]).wait()
        didx = dst_v.at[g]
        pltpu.async_copy(bufs[b], acc_sh.at[didx], sems[b], add=True)
        if deg_sh is not None:
            pltpu.async_copy(ones_v, deg_sh.at[didx], dsem, add=True)
        if g + 2 < NG:
            b2 = (g + 2) % 3
            if g >= 1:
                pltpu.make_async_copy(
                    bufs[b2], acc_sh.at[didx], sems[b2]).wait()
            pltpu.async_copy(tab_sh.at[src_v.at[g + 2]], bufs[b2], sems[b2])
    for t in (NG - 2, NG - 1):
        pltpu.make_async_copy(bufs[t % 3], acc_sh.at[dst_v.at[t]], sems[t % 3]).wait()
    if deg_sh is not None:
        for t in range(NG):
            pltpu.make_async_copy(ones_v, deg_sh.at[dst_v.at[t]], dsem).wait()


def _sc_seg_deg_kernel(p_hbm, src_hbm, dst_hbm, z32_hbm, z1_hbm,
                       out_hbm, deg_hbm,
                       src_v, dst_v, rows_a, rows_b, rows_c, ones_v,
                       acc_sh, deg_sh, tab_sh, gsem, ssem, xsem, dsem):
    c = lax.axis_index("c")
    s = lax.axis_index("s")
    wid = s * 2 + c
    pltpu.sync_copy(z32_hbm, acc_sh.at[pl.ds(s * RPT, RPT)])
    pltpu.sync_copy(z1_hbm, deg_sh.at[pl.ds(s * RPT, RPT)])
    for k in range(GB * CH // 16):
        ones_v[pl.ds(k * 16, 16)] = jnp.ones((16,), _f32)
    pltpu.sync_copy(src_hbm.at[pl.ds(wid * NG, NG)], src_v)
    pltpu.sync_copy(dst_hbm.at[pl.ds(wid * NG, NG)], dst_v)
    pltpu.sync_copy(p_hbm.at[pl.ds(s * RPT, RPT)], tab_sh.at[pl.ds(s * RPT, RPT)])
    plsc.subcore_barrier()

    _seg_loop(tab_sh, acc_sh, src_v, dst_v, (rows_a, rows_b, rows_c),
              (gsem, ssem, xsem), deg_sh=deg_sh, ones_v=ones_v, dsem=dsem)

    plsc.subcore_barrier()
    pltpu.sync_copy(acc_sh.at[pl.ds(s * RPT, RPT)],
                    out_hbm.at[pl.ds(c * NPAD + s * RPT, RPT)])
    pltpu.sync_copy(deg_sh.at[pl.ds(s * RPT, RPT)],
                    deg_hbm.at[pl.ds(c * NPAD + s * RPT, RPT)])


def _sc_seg_kernel(p_hbm, src_hbm, dst_hbm, z32_hbm,
                   out_hbm,
                   src_v, dst_v, rows_a, rows_b, rows_c,
                   acc_sh, tab_sh, gsem, ssem, xsem):
    c = lax.axis_index("c")
    s = lax.axis_index("s")
    wid = s * 2 + c
    pltpu.sync_copy(z32_hbm, acc_sh.at[pl.ds(s * RPT, RPT)])
    pltpu.sync_copy(src_hbm.at[pl.ds(wid * NG, NG)], src_v)
    pltpu.sync_copy(dst_hbm.at[pl.ds(wid * NG, NG)], dst_v)
    pltpu.sync_copy(p_hbm.at[pl.ds(s * RPT, RPT)], tab_sh.at[pl.ds(s * RPT, RPT)])
    plsc.subcore_barrier()

    _seg_loop(tab_sh, acc_sh, src_v, dst_v, (rows_a, rows_b, rows_c),
              (gsem, ssem, xsem))

    plsc.subcore_barrier()
    pltpu.sync_copy(acc_sh.at[pl.ds(s * RPT, RPT)],
                    out_hbm.at[pl.ds(c * NPAD + s * RPT, RPT)])


def _sc_seg_deg(p, src_r, dst_r, z32, z1, h):
    fn = functools.partial(
        pl.kernel,
        out_type=[
            jax.ShapeDtypeStruct((2 * NPAD, h), _f32),
            jax.ShapeDtypeStruct((2 * NPAD,), _f32),
        ],
        mesh=_sc_mesh(),
        compiler_params=pltpu.CompilerParams(use_tc_tiling_on_sc=False),
        scratch_types=[
            pltpu.VMEM((NG, GB * CH), jnp.int32),
            pltpu.VMEM((NG, GB * CH), jnp.int32),
            pltpu.VMEM((GB * CH, h), _f32),
            pltpu.VMEM((GB * CH, h), _f32),
            pltpu.VMEM((GB * CH, h), _f32),
            pltpu.VMEM((GB * CH,), _f32),
            pltpu.VMEM_SHARED((NPAD, h), _f32),
            pltpu.VMEM_SHARED((NPAD,), _f32),
            pltpu.VMEM_SHARED((NPAD, h), _f32),
            pltpu.SemaphoreType.DMA,
            pltpu.SemaphoreType.DMA,
            pltpu.SemaphoreType.DMA,
            pltpu.SemaphoreType.DMA,
        ],
    )(_sc_seg_deg_kernel)
    return fn(p, src_r, dst_r, z32, z1)


def _sc_seg(p, src_r, dst_r, z32, h):
    fn = functools.partial(
        pl.kernel,
        out_type=jax.ShapeDtypeStruct((2 * NPAD, h), _f32),
        mesh=_sc_mesh(),
        compiler_params=pltpu.CompilerParams(use_tc_tiling_on_sc=False),
        scratch_types=[
            pltpu.VMEM((NG, GB * CH), jnp.int32),
            pltpu.VMEM((NG, GB * CH), jnp.int32),
            pltpu.VMEM((GB * CH, h), _f32),
            pltpu.VMEM((GB * CH, h), _f32),
            pltpu.VMEM((GB * CH, h), _f32),
            pltpu.VMEM_SHARED((NPAD, h), _f32),
            pltpu.VMEM_SHARED((NPAD, h), _f32),
            pltpu.SemaphoreType.DMA,
            pltpu.SemaphoreType.DMA,
            pltpu.SemaphoreType.DMA,
        ],
    )(_sc_seg_kernel)
    return fn(p, src_r, dst_r, z32)


def kernel(x, edge_index, W1_l, b1, W1_r, W2_l, b2, W2_r, w):
    h1 = W1_l.shape[1]
    h2 = W2_l.shape[1]
    src = edge_index[0].astype(jnp.int32)
    dst = edge_index[1].astype(jnp.int32)
    epad = EPAD - N_EDGES
    src_r = jnp.concatenate([src, jnp.zeros((epad,), jnp.int32)]).reshape(-1, GB * CH)
    dst_r = jnp.concatenate([dst, jnp.full((epad,), N_NODES, jnp.int32)]).reshape(-1, GB * CH)
    x_pad = jnp.concatenate([x, jnp.zeros((NPAD - N_NODES, D_IN), _f32)])
    z32 = jnp.zeros((RPT, h1), _f32)
    z1 = jnp.zeros((RPT,), _f32)

    p1, s1 = _tc1(x_pad, W1_l, W1_r, b1)
    part1, degp = _sc_seg_deg(p1, src_r, dst_r, z32, z1, h1)
    p2, s2, inv = _tc2(part1, degp.reshape(2 * NPAD, 1), s1, W2_l, W2_r, b2)
    part2 = _sc_seg(p2, src_r, dst_r, z32, h2)
    out_pad = _tc3(part2, inv, s2, w)
    return out_pad[:N_NODES]
